# Initial kernel scaffold; baseline (speedup 1.0000x reference)
import jax
import jax.numpy as jnp
from jax.experimental import pallas as pl


def kernel(points, xyz):
    raise NotImplementedError("write your pallas kernel here")


# speedup vs baseline: 1.0043x; 1.0043x over previous
import functools

import jax
import jax.numpy as jnp
from jax.experimental import pallas as pl
from jax.experimental.pallas import tpu as pltpu

NPOINT = 6144


def _match_body(a3_ref, bt_ref, nmax_ref, nidx_ref, *, nj):
    a = a3_ref[0]
    bt = bt_ref[0]

    pa = a * a
    n2a = (pa[:, 0:1] + pa[:, 1:2]) + pa[:, 2:3]
    abf = (a / jnp.sqrt(n2a)).astype(jnp.bfloat16)
    pb = bt * bt
    n2b = (pb[0:1, :] + pb[1:2, :]) + pb[2:3, :]
    bbf = (bt / jnp.sqrt(n2b)).astype(jnp.bfloat16)

    s = jax.lax.dot_general(
        abf, bbf, (((1,), (0,)), ((), ())),
        preferred_element_type=jnp.float32)

    nmax = jnp.max(s, axis=1)
    jiota = jax.lax.broadcasted_iota(jnp.int32, s.shape, 1)
    cand = jnp.where(s == nmax[:, None], jiota, nj)
    nidx = jnp.min(cand, axis=1)
    nmax_ref[0, 0, :] = nmax
    nidx_ref[0, 0, :] = nidx


def _node_max_idx(xyz):
    n, t, _ = xyz.shape
    nj = t // 2
    bi = 512
    nblk = nj // bi
    a3 = xyz[:, ::2, :]
    bt = jnp.transpose(xyz[:, 1::2, :], (0, 2, 1))
    grid = (n, nblk)
    nmax, nidx = pl.pallas_call(
        functools.partial(_match_body, nj=nj),
        grid=grid,
        in_specs=[
            pl.BlockSpec((1, bi, 3), lambda b, i: (b, i, 0)),
            pl.BlockSpec((1, 3, nj), lambda b, i: (b, 0, 0)),
        ],
        out_specs=[
            pl.BlockSpec((1, 1, bi), lambda b, i: (b * nblk + i, 0, 0)),
            pl.BlockSpec((1, 1, bi), lambda b, i: (b * nblk + i, 0, 0)),
        ],
        out_shape=[
            jax.ShapeDtypeStruct((n * nblk, 1, bi), jnp.float32),
            jax.ShapeDtypeStruct((n * nblk, 1, bi), jnp.int32),
        ],
    )(a3, bt)
    return nmax.reshape(n, nj), nidx.reshape(n, nj)


def _merge_like(x, unm_idx, src_idx, dst_idx):
    src = x[:, ::2, :]
    dst = x[:, 1::2, :]
    unm = jnp.take_along_axis(src, unm_idx[:, :, None], axis=1)
    srcg = jnp.take_along_axis(src, src_idx[:, :, None], axis=1)

    def _scatter_mean(d, i, s):
        sums = d.at[i].add(s)
        counts = jnp.ones((d.shape[0],), dtype=d.dtype).at[i].add(1.0)
        return sums / counts[:, None]

    dst = jax.vmap(_scatter_mean)(dst, dst_idx, srcg)
    return jnp.concatenate([unm, dst], axis=1)


def _unmerge_like(x, unm_idx, src_idx, dst_idx, t):
    unm_len = unm_idx.shape[1]
    unm = x[:, :unm_len, :]
    dst = x[:, unm_len:, :]
    src = jnp.take_along_axis(dst, dst_idx[:, :, None], axis=1)
    n = x.shape[0]
    c = x.shape[2]
    out = jnp.zeros((n, t, c), dtype=x.dtype)
    out = out.at[:, 1::2, :].set(dst)

    def _scat(o, ui, u, si, s):
        o = o.at[2 * ui].set(u)
        o = o.at[2 * si].set(s)
        return o

    return jax.vmap(_scat)(out, unm_idx, unm, src_idx, src)


def kernel(points, xyz):
    t = points.shape[1]
    r = t - NPOINT
    r = min(r, t // 2)
    node_max, node_idx = _node_max_idx(xyz)
    edge_idx = jnp.argsort(-node_max, axis=-1)
    unm_idx = edge_idx[:, r:]
    src_idx = edge_idx[:, :r]
    dst_idx = jnp.take_along_axis(node_idx, src_idx, axis=-1)
    merged_pts = _merge_like(points, unm_idx, src_idx, dst_idx)
    merged_xyz = _merge_like(xyz, unm_idx, src_idx, dst_idx)
    compressed_pts = _unmerge_like(merged_pts, unm_idx, src_idx, dst_idx, t)
    compressed_xyz = _unmerge_like(merged_xyz, unm_idx, src_idx, dst_idx, t)
    return (merged_pts, compressed_pts, merged_xyz, compressed_xyz)
